# baseline (speedup 1.0000x reference)
import functools
from types import SimpleNamespace

import numpy as np
import jax
import jax.numpy as jnp
from jax.experimental import pallas as pl
from jax.experimental.pallas import tpu as pltpu


def _lrelu(y, slope):
    return jnp.where(y >= 0.0, y, slope * y)


def _quadrant_flats3(P, Hq, Wh, C):
    out = []
    for a in range(2):
        Pa = P.reshape(Hq, 2, 2 * Wh, C)[:, a]
        for b in range(2):
            Q = Pa.reshape(Hq, Wh, 2, C)[:, :, b, :]
            out.append(Q.reshape(Hq * Wh, C))
    return out


def _quadrant_flats_flat(v, Hp, Wp, C):
    out = []
    for a in range(2):
        for b in range(2):
            t = v.reshape(Hp * Wp // 2, 2, C)[:, b, :]
            t = t.reshape(Hp // 2, 2, Wp // 2, C)[:, a]
            out.append(t.reshape((Hp // 2) * (Wp // 2), C))
    return out


def _s2_conv_raw(qs, Wh, p_out, w_ref, b_ref, slope):
    xcat = jnp.concatenate(
        [qs[(dy % 2) * 2 + (dx % 2)][(dy // 2) * Wh + dx // 2:
                                     (dy // 2) * Wh + dx // 2 + p_out, :]
         for dy in range(4) for dx in range(4)], axis=1)
    y = jnp.dot(xcat, w_ref[...], preferred_element_type=jnp.float32)
    return _lrelu(y + b_ref[0], slope).astype(jnp.bfloat16)


def _scatter_pad3(dst3, y, Wi, Ho, Wo):
    dst3[...] = jnp.zeros(dst3.shape, dst3.dtype)
    for r in range(Ho):
        dst3[r + 2, 2:2 + Wo, :] = y[r * Wi:r * Wi + Wo, :]


def _scale_body(x_ref, w0, b0, w1, b1, w2, b2, w3, b3, w4, b4, o_ref,
                P0, P1, P2, X3, X4, *, D, slope=0.2):
    if P0 is None:
        v_in = x_ref[...]
    else:
        S, Wp0 = D.S, D.Wp0
        P0[...] = jnp.zeros(P0.shape, P0.dtype)
        for r in range(S):
            P0[(r + 2) * Wp0 + 2:(r + 2) * Wp0 + 2 + S, :] = \
                x_ref[r * S:r * S + S, :]
        v_in = P0[...]

    p0_out = D.Hk0 * D.Wh0
    qs0 = _quadrant_flats_flat(v_in, D.S + 6, D.Wp0, 3)
    xc0 = jnp.concatenate(
        [qs0[g][s:s + p0_out, :] for s in (0, 1, D.Wh0, D.Wh0 + 1)
         for g in range(4)], axis=1)
    y0 = jnp.dot(xc0, w0[...], preferred_element_type=jnp.float32)
    v0 = _lrelu(y0 + b0[0], slope).astype(jnp.bfloat16)

    _scatter_pad3(P1, v0, D.Wh0, D.Ho0, D.Ho0)
    qs1 = _quadrant_flats3(P1[...], D.Hq1, D.Wh1, 64)
    v1 = _s2_conv_raw(qs1, D.Wh1, D.p1_out, w1, b1, slope)

    _scatter_pad3(P2, v1, D.Wh1, D.Ho1, D.Ho1)
    qs2 = _quadrant_flats3(P2[...], D.Hq2, D.Wh2, 128)
    v2 = _s2_conv_raw(qs2, D.Wh2, D.p2_out, w2, b2, slope)

    X3[...] = jnp.zeros(X3.shape, X3.dtype)
    for r in range(D.Ho2):
        X3[(r + 2) * D.Wi3 + 2:(r + 2) * D.Wi3 + 2 + D.Ho2, :] = \
            v2[r * D.Wh2:r * D.Wh2 + D.Ho2, :]
    x3 = X3[...]
    p3_out = D.Ho3 * D.Wi3
    shifts3 = tuple(dy * D.Wi3 + dx for dy in range(4) for dx in range(4))
    xc3 = jnp.concatenate([x3[s:s + p3_out, :] for s in shifts3], axis=1)
    y3 = jnp.dot(xc3, w3[...], preferred_element_type=jnp.float32)
    y3 = _lrelu(y3 + b3[0], slope).astype(jnp.bfloat16)

    wi4 = D.Wo3 + 4
    X4[...] = jnp.zeros(X4.shape, X4.dtype)
    for r in range(D.Ho3):
        X4[(r + 2) * wi4 + 2:(r + 2) * wi4 + 2 + D.Wo3, :] = \
            y3[r * D.Wi3:r * D.Wi3 + D.Wo3, :]
    p4_out = (D.Ho3 + 1) * wi4
    t2 = jax.lax.dot_general(w4[...], X4[...], (((1,), (1,)), ((), ())),
                             preferred_element_type=jnp.float32)
    acc4 = None
    for t, s in enumerate(dy * wi4 + dx for dy in range(4) for dx in range(4)):
        part = t2[t:t + 1, s:s + p4_out]
        acc4 = part if acc4 is None else acc4 + part
    y4 = acc4 + b4[0, 0]
    o_ref[...] = jnp.concatenate(
        [y4[:, r * wi4:r * wi4 + D.Wo4] for r in range(D.Ho4)], axis=0)


def _dims(S):
    D = SimpleNamespace()
    D.S = S
    D.Wp0 = S + 4
    D.p0_flat = (S + 6) * (S + 4)
    D.Hh0 = (S + 4) // 2
    D.Wh0 = D.Hh0
    D.Hk0 = D.Hh0 - 1
    D.Ho0 = S // 2 + 1
    D.Hq1 = (D.Ho0 + 5) // 2 + 1
    D.Wh1 = (D.Ho0 + 5) // 2
    D.p1_out = (D.Wh1 - 1) * D.Wh1
    D.Ho1 = D.Ho0 // 2 + 1
    D.Hq2 = (D.Ho1 + 5) // 2 + 1
    D.Wh2 = (D.Ho1 + 5) // 2
    D.p2_out = (D.Wh2 - 1) * D.Wh2
    D.Ho2 = D.Ho1 // 2 + 1
    D.Wi3 = D.Ho2 + 4
    D.p3_in = (D.Ho2 + 5) * D.Wi3
    D.Ho3 = D.Ho2 + 1
    D.Wo3 = D.Ho2 + 1
    D.wi4 = D.Wo3 + 4
    D.p4_in = (D.Ho3 + 5) * D.wi4
    D.p4_out = (D.Ho3 + 1) * D.wi4
    D.Ho4 = D.Ho3 + 1
    D.Wo4 = D.Wo3 + 1
    return D


def _w0_s2(w):
    return (w.reshape(2, 2, 2, 2, 3, 64)
             .transpose(0, 2, 1, 3, 4, 5)
             .reshape(48, 64).astype(jnp.bfloat16))


def _tri_body(*refs, DS):
    xs = refs[0:3]
    outs = refs[33:36]
    scr = refs[36:]
    si = 0
    for k in range(3):
        wb = refs[3 + 10 * k:13 + 10 * k]
        if k == 0:
            P0, nscr = None, 4
        else:
            P0, nscr = scr[si], 5
            si += 1
        _scale_body(xs[k], *wb, outs[k], P0, *scr[si:si + 4], D=DS[k])
        si += 4


def _run_scales(xs_flat, sizes, Ws, Bs):
    N = xs_flat[0].shape[0]
    DS = [_dims(s) for s in sizes]

    operands, in_specs = [], []
    for xf in xs_flat:
        operands.append(xf)
        in_specs.append(
            pl.BlockSpec((None,) + xf.shape[1:], lambda n: (n, 0, 0)))
    for k in range(3):
        ws, bs = Ws[k], Bs[k]
        packed = [_w0_s2(ws[0]), bs[0].reshape(1, -1),
                  ws[1].reshape(16 * 64, 128).astype(jnp.bfloat16),
                  bs[1].reshape(1, -1),
                  ws[2].reshape(16 * 128, 256).astype(jnp.bfloat16),
                  bs[2].reshape(1, -1),
                  ws[3].reshape(16 * 256, 512).astype(jnp.bfloat16),
                  bs[3].reshape(1, -1),
                  ws[4].reshape(16, 512).astype(jnp.bfloat16),
                  bs[4].reshape(1, 1)]
        for a in packed:
            operands.append(a)
            in_specs.append(pl.BlockSpec(a.shape, lambda n: (0, 0)))

    out_shapes = tuple(jax.ShapeDtypeStruct((N, D.Ho4, D.Wo4), jnp.float32)
                       for D in DS)
    out_specs = tuple(pl.BlockSpec((None, D.Ho4, D.Wo4), lambda n: (n, 0, 0))
                      for D in DS)
    scratch = []
    for k, D in enumerate(DS):
        if k > 0:
            scratch.append(pltpu.VMEM((D.p0_flat, 3), jnp.bfloat16))
        scratch += [pltpu.VMEM((2 * D.Hq1, 2 * D.Wh1, 64), jnp.bfloat16),
                    pltpu.VMEM((2 * D.Hq2, 2 * D.Wh2, 128), jnp.bfloat16),
                    pltpu.VMEM((D.p3_in, 256), jnp.bfloat16),
                    pltpu.VMEM((D.p4_in, 512), jnp.bfloat16)]

    outs = pl.pallas_call(
        functools.partial(_tri_body, DS=DS),
        out_shape=out_shapes,
        grid=(N,),
        in_specs=in_specs,
        out_specs=out_specs,
        scratch_shapes=scratch,
        compiler_params=pltpu.CompilerParams(
            dimension_semantics=("parallel",)),
    )(*operands)
    return [o[..., None] for o in outs]


def _pool1d(n):
    no = (n - 1) // 2 + 1
    p = np.zeros((no, n), np.float32)
    for o in range(no):
        cols = [c for c in (2 * o - 1, 2 * o, 2 * o + 1) if 0 <= c < n]
        p[o, cols] = 1.0 / len(cols)
    return p


def _pools_kernel(m1_ref, m2_ref, x_ref, o1_ref, o2_ref, *, N, C):
    p1 = jnp.dot(m1_ref[...], x_ref[...], preferred_element_type=jnp.float32)
    p1b = p1.astype(jnp.bfloat16)
    p2 = jnp.dot(m2_ref[...], p1b, preferred_element_type=jnp.float32)
    p2b = p2.astype(jnp.bfloat16)
    for n in range(N):
        o1_ref[n, :, :] = p1b[:, C * n:C * n + C]
        o2_ref[n, :, :] = p2b[:, C * n:C * n + C]


def _pools(x):
    N, H, W, C = x.shape
    m1 = jnp.asarray(np.kron(_pool1d(H), _pool1d(W)), dtype=jnp.bfloat16)
    H2 = (H - 1) // 2 + 1
    m2 = jnp.asarray(np.kron(_pool1d(H2), _pool1d(H2)), dtype=jnp.bfloat16)
    xt = x.transpose(1, 2, 0, 3).reshape(H * W, N * C).astype(jnp.bfloat16)
    lanes = N * C
    H3 = (H2 - 1) // 2 + 1
    return pl.pallas_call(
        functools.partial(_pools_kernel, N=N, C=C),
        out_shape=(jax.ShapeDtypeStruct((N, H2 * H2, C), jnp.bfloat16),
                   jax.ShapeDtypeStruct((N, H3 * H3, C), jnp.bfloat16)),
        grid=(1,),
        in_specs=[
            pl.BlockSpec(m1.shape, lambda i: (0, 0)),
            pl.BlockSpec(m2.shape, lambda i: (0, 0)),
            pl.BlockSpec((H * W, lanes), lambda i: (0, 0)),
        ],
        out_specs=(pl.BlockSpec((N, H2 * H2, C), lambda i: (0, 0, 0)),
                   pl.BlockSpec((N, H3 * H3, C), lambda i: (0, 0, 0))),
        compiler_params=pltpu.CompilerParams(
            dimension_semantics=("arbitrary",)),
    )(m1, m2, xt)


def kernel(x, w_0_0, b_0_0, w_0_1, b_0_1, w_0_2, b_0_2, w_0_3, b_0_3, w_0_4, b_0_4,
           w_1_0, b_1_0, w_1_1, b_1_1, w_1_2, b_1_2, w_1_3, b_1_3, w_1_4, b_1_4,
           w_2_0, b_2_0, w_2_1, b_2_1, w_2_2, b_2_2, w_2_3, b_2_3, w_2_4, b_2_4):
    Ws = [[w_0_0, w_0_1, w_0_2, w_0_3, w_0_4],
          [w_1_0, w_1_1, w_1_2, w_1_3, w_1_4],
          [w_2_0, w_2_1, w_2_2, w_2_3, w_2_4]]
    Bs = [[b_0_0, b_0_1, b_0_2, b_0_3, b_0_4],
          [b_1_0, b_1_1, b_1_2, b_1_3, b_1_4],
          [b_2_0, b_2_1, b_2_2, b_2_3, b_2_4]]
    N, S = x.shape[0], x.shape[1]
    x2f, x3f = _pools(x)
    xa = jnp.pad(x, ((0, 0), (2, 4), (2, 2), (0, 0)))
    xa = xa.reshape(N, (S + 6) * (S + 4), 3).astype(jnp.bfloat16)
    return _run_scales([xa, x2f, x3f], [S, S // 2, S // 4],
                       [Ws[2], Ws[1], Ws[0]],
                       [Bs[2], Bs[1], Bs[0]])


# speedup vs baseline: 1.4613x; 1.4613x over previous
import functools
from types import SimpleNamespace

import numpy as np
import jax
import jax.numpy as jnp
from jax.experimental import pallas as pl
from jax.experimental.pallas import tpu as pltpu


def _lrelu(y, slope):
    return jnp.where(y >= 0.0, y, slope * y)


def _quadrant_flats(P, Hq, Wh, C):
    out = []
    for a in range(2):
        Pa = P.reshape(Hq, 2, 2 * Wh, C)[:, a]
        for b in range(2):
            Q = Pa.reshape(Hq, Wh, 2, C)[:, :, b, :]
            out.append(Q.reshape(Hq * Wh, C))
    return out


def _s2_conv_block(P, Hq, Wh, C, p_out, w_ref, b_ref, slope, out_dtype):
    qs = _quadrant_flats(P, Hq, Wh, C)
    xcat = jnp.concatenate(
        [qs[(dy % 2) * 2 + (dx % 2)][(dy // 2) * Wh + dx // 2:
                                     (dy // 2) * Wh + dx // 2 + p_out, :]
         for dy in range(4) for dx in range(4)], axis=1)
    y = jnp.dot(xcat, w_ref[...], preferred_element_type=jnp.float32)
    y = _lrelu(y + b_ref[0], slope)
    return y.astype(out_dtype)


def _scatter_pad3(dst3, y, Wi, Ho, Wo):
    dst3[...] = jnp.zeros(dst3.shape, dst3.dtype)
    for r in range(Ho):
        dst3[r + 2, 2:2 + Wo, :] = y[r * Wi:r * Wi + Wo, :]


def _scale_body(x_ref, w0, b0, w1, b1, w2, b2, w3, b3, w4, b4, o_ref,
                P1, P2, X3, X4, *, D, quad_in, slope=0.2):
    p0_out = D.Hk0 * D.Wh0
    if quad_in:
        v = x_ref[...]
        xc0 = jnp.concatenate(
            [v[g, s:s + p0_out, :] for s in (0, 1, D.Wh0, D.Wh0 + 1)
             for g in range(4)], axis=1)
        y0 = jnp.dot(xc0, w0[...], preferred_element_type=jnp.float32)
    else:
        y0 = None
        for t, s in enumerate((0, 1, D.Wh0, D.Wh0 + 1)):
            part = jnp.dot(x_ref[s:s + p0_out, :], w0[t * 12:(t + 1) * 12, :],
                           preferred_element_type=jnp.float32)
            y0 = part if y0 is None else y0 + part
    v0 = _lrelu(y0 + b0[0], slope).astype(jnp.bfloat16)

    _scatter_pad3(P1, v0, D.Wh0, D.Ho0, D.Ho0)
    v1 = _s2_conv_block(P1[...], D.Hq1, D.Wh1, 64, D.p1_out, w1, b1, slope,
                        jnp.bfloat16)

    _scatter_pad3(P2, v1, D.Wh1, D.Ho1, D.Ho1)
    v2 = _s2_conv_block(P2[...], D.Hq2, D.Wh2, 128, D.p2_out, w2, b2, slope,
                        jnp.float32)

    X3[...] = jnp.zeros(X3.shape, X3.dtype)
    for r in range(D.Ho2):
        X3[(r + 2) * D.Wi3 + 2:(r + 2) * D.Wi3 + 2 + D.Ho2, :] = \
            v2[r * D.Wh2:r * D.Wh2 + D.Ho2, :]
    x3 = X3[...]
    p3_out = D.Ho3 * D.Wi3
    shifts3 = tuple(dy * D.Wi3 + dx for dy in range(4) for dx in range(4))
    xc3 = jnp.concatenate([x3[s:s + p3_out, :] for s in shifts3], axis=1)
    y3 = jnp.dot(xc3, w3[...], preferred_element_type=jnp.float32)
    y3 = _lrelu(y3 + b3[0], slope).astype(jnp.bfloat16)

    wi4 = D.Wo3 + 4
    X4[...] = jnp.zeros(X4.shape, X4.dtype)
    for r in range(D.Ho3):
        X4[(r + 2) * wi4 + 2:(r + 2) * wi4 + 2 + D.Wo3, :] = \
            y3[r * D.Wi3:r * D.Wi3 + D.Wo3, :]
    p4_out = (D.Ho3 + 1) * wi4
    t2 = jax.lax.dot_general(w4[...], X4[...], (((1,), (1,)), ((), ())),
                             preferred_element_type=jnp.float32)
    acc4 = None
    for t, s in enumerate(dy * wi4 + dx for dy in range(4) for dx in range(4)):
        part = t2[t:t + 1, s:s + p4_out]
        acc4 = part if acc4 is None else acc4 + part
    y4 = acc4 + b4[0, 0]
    o_ref[...] = jnp.concatenate(
        [y4[:, r * wi4:r * wi4 + D.Wo4] for r in range(D.Ho4)], axis=0)


def _dims(S):
    D = SimpleNamespace()
    D.S = S
    D.Hh0 = (S + 4) // 2
    D.Wh0 = D.Hh0
    D.Hq0 = D.Hh0 + 1
    D.Hk0 = D.Hh0 - 1
    D.Ho0 = S // 2 + 1
    D.Hq1 = (D.Ho0 + 5) // 2 + 1
    D.Wh1 = (D.Ho0 + 5) // 2
    D.p1_out = (D.Wh1 - 1) * D.Wh1
    D.Ho1 = D.Ho0 // 2 + 1
    D.Hq2 = (D.Ho1 + 5) // 2 + 1
    D.Wh2 = (D.Ho1 + 5) // 2
    D.p2_out = (D.Wh2 - 1) * D.Wh2
    D.Ho2 = D.Ho1 // 2 + 1
    D.Wi3 = D.Ho2 + 4
    D.p3_in = (D.Ho2 + 5) * D.Wi3
    D.Ho3 = D.Ho2 + 1
    D.Wo3 = D.Ho2 + 1
    D.wi4 = D.Wo3 + 4
    D.p4_in = (D.Ho3 + 5) * D.wi4
    D.p4_out = (D.Ho3 + 1) * D.wi4
    D.Ho4 = D.Ho3 + 1
    D.Wo4 = D.Wo3 + 1
    return D


def _prep_conv0(x):
    N, H, W, Cin = x.shape
    xp = jnp.pad(x, ((0, 0), (2, 2), (2, 2), (0, 0)))
    Hp = xp.shape[1]
    xin = xp.reshape(N, Hp // 2, 2, Hp // 2, 2, Cin)
    xin = xin.transpose(0, 1, 3, 2, 4, 5).reshape(N, Hp // 2, Hp // 2, 4 * Cin)
    xin = jnp.pad(xin, ((0, 0), (0, 1), (0, 0), (0, 0)))
    Hh = Hp // 2
    return xin.reshape(N, (Hh + 1) * Hh, 4 * Cin).astype(jnp.bfloat16)


def _w_s2(w):
    cin, cout = w.shape[2], w.shape[3]
    return (w.reshape(2, 2, 2, 2, cin, cout)
             .transpose(0, 2, 1, 3, 4, 5)
             .reshape(16 * cin, cout))


def _pool1d(n):
    no = (n - 1) // 2 + 1
    p = np.zeros((no, n), np.float32)
    for o in range(no):
        cols = [c for c in (2 * o - 1, 2 * o, 2 * o + 1) if 0 <= c < n]
        p[o, cols] = 1.0 / len(cols)
    return p


def _quad_pool_matrix(m, s):
    wh = (s + 4) // 2
    hq = wh + 1
    out = np.zeros((4 * hq * wh, m.shape[1]), np.float32)
    for a in range(2):
        for b in range(2):
            g = a * 2 + b
            for i in range(hq):
                for j in range(wh):
                    h, w = 2 * i + a - 2, 2 * j + b - 2
                    if 0 <= h < s and 0 <= w < s:
                        out[g * hq * wh + i * wh + j] = m[h * s + w]
    return out


def _pools_kernel(m1_ref, m1q_ref, m2q_ref, x_ref, *wio, N, C, pqB, pqC):
    win, wout = wio[:12], wio[14:]
    qB_ref, qC_ref = wio[12], wio[13]
    p1 = jnp.dot(m1_ref[...], x_ref[...], preferred_element_type=jnp.float32)
    p1b = p1.astype(jnp.bfloat16)
    qb = jnp.dot(m1q_ref[...], x_ref[...],
                 preferred_element_type=jnp.float32).astype(jnp.bfloat16)
    qc = jnp.dot(m2q_ref[...], p1b,
                 preferred_element_type=jnp.float32).astype(jnp.bfloat16)
    for n in range(N):
        qB_ref[n] = qb[:, C * n:C * n + C].reshape(4, pqB, C)
        qC_ref[n] = qc[:, C * n:C * n + C].reshape(4, pqC, C)
    for i in range(12):
        wout[i][...] = win[i][...].astype(jnp.bfloat16)


def _pools_and_weights(x, Ws, Bs):
    N, H, W, C = x.shape
    m1np = np.kron(_pool1d(H), _pool1d(W))
    H2 = (H - 1) // 2 + 1
    m2np = np.kron(_pool1d(H2), _pool1d(H2))
    m1 = jnp.asarray(m1np, dtype=jnp.bfloat16)
    m1q = jnp.asarray(_quad_pool_matrix(m1np, H2), dtype=jnp.bfloat16)
    m2q = jnp.asarray(_quad_pool_matrix(m2np, (H2 - 1) // 2 + 1),
                      dtype=jnp.bfloat16)
    xt = x.transpose(1, 2, 0, 3).reshape(H * W, N * C).astype(jnp.bfloat16)

    DB, DC = _dims(H2), _dims((H2 - 1) // 2 + 1)
    pqB, pqC = DB.Hq0 * DB.Wh0, DC.Hq0 * DC.Wh0

    w_f32 = []
    for k in range(3):
        ws = Ws[k]
        w_f32 += [_w_s2(ws[0]),
                  ws[1].reshape(16 * 64, 128),
                  ws[2].reshape(16 * 128, 256),
                  ws[4].reshape(16, 512)]

    out_shape = ([jax.ShapeDtypeStruct((N, 4, pqB, C), jnp.bfloat16),
                  jax.ShapeDtypeStruct((N, 4, pqC, C), jnp.bfloat16)]
                 + [jax.ShapeDtypeStruct(w.shape, jnp.bfloat16)
                    for w in w_f32])
    in_specs = ([pl.BlockSpec(m1.shape, lambda i: (0, 0)),
                 pl.BlockSpec(m1q.shape, lambda i: (0, 0)),
                 pl.BlockSpec(m2q.shape, lambda i: (0, 0)),
                 pl.BlockSpec(xt.shape, lambda i: (0, 0))]
                + [pl.BlockSpec(w.shape, lambda i: (0, 0)) for w in w_f32])
    out_specs = ([pl.BlockSpec((N, 4, pqB, C), lambda i: (0, 0, 0, 0)),
                  pl.BlockSpec((N, 4, pqC, C), lambda i: (0, 0, 0, 0))]
                 + [pl.BlockSpec(w.shape, lambda i: (0, 0)) for w in w_f32])

    outs = pl.pallas_call(
        functools.partial(_pools_kernel, N=N, C=C, pqB=pqB, pqC=pqC),
        out_shape=tuple(out_shape),
        grid=(1,),
        in_specs=in_specs,
        out_specs=tuple(out_specs),
        compiler_params=pltpu.CompilerParams(
            dimension_semantics=("arbitrary",)),
    )(m1, m1q, m2q, xt, *w_f32)
    return outs[0], outs[1], list(outs[2:])


def _tri_body(*refs, DS):
    xs = refs[0:3]
    outs = refs[33:36]
    scr = refs[36:]
    for k in range(3):
        wb = refs[3 + 10 * k:13 + 10 * k]
        _scale_body(xs[k], *wb, outs[k], *scr[4 * k:4 * k + 4],
                    D=DS[k], quad_in=(k > 0))


def _run_scales(xa_flat, qB, qC, wb16, W3s, Bs):
    N = xa_flat.shape[0]
    sizes = [64, 32, 16]
    DS = [_dims(s) for s in sizes]

    operands = [xa_flat, qB, qC]
    in_specs = [
        pl.BlockSpec((None,) + xa_flat.shape[1:], lambda n: (n, 0, 0)),
        pl.BlockSpec((None,) + qB.shape[1:], lambda n: (n, 0, 0, 0)),
        pl.BlockSpec((None,) + qC.shape[1:], lambda n: (n, 0, 0, 0)),
    ]
    for k in range(3):
        bs = Bs[k]
        packed = [wb16[4 * k + 0], bs[0].reshape(1, -1),
                  wb16[4 * k + 1], bs[1].reshape(1, -1),
                  wb16[4 * k + 2], bs[2].reshape(1, -1),
                  W3s[k].reshape(16 * 256, 512), bs[3].reshape(1, -1),
                  wb16[4 * k + 3], bs[4].reshape(1, 1)]
        for a in packed:
            operands.append(a)
            in_specs.append(pl.BlockSpec(a.shape, lambda n: (0, 0)))

    out_shapes = tuple(jax.ShapeDtypeStruct((N, D.Ho4, D.Wo4), jnp.float32)
                       for D in DS)
    out_specs = tuple(pl.BlockSpec((None, D.Ho4, D.Wo4), lambda n: (n, 0, 0))
                      for D in DS)
    scratch = []
    for D in DS:
        scratch += [pltpu.VMEM((2 * D.Hq1, 2 * D.Wh1, 64), jnp.bfloat16),
                    pltpu.VMEM((2 * D.Hq2, 2 * D.Wh2, 128), jnp.bfloat16),
                    pltpu.VMEM((D.p3_in, 256), jnp.float32),
                    pltpu.VMEM((D.p4_in, 512), jnp.bfloat16)]

    outs = pl.pallas_call(
        functools.partial(_tri_body, DS=DS),
        out_shape=out_shapes,
        grid=(N,),
        in_specs=in_specs,
        out_specs=out_specs,
        scratch_shapes=scratch,
        compiler_params=pltpu.CompilerParams(
            dimension_semantics=("parallel",)),
    )(*operands)
    return [o[..., None] for o in outs]


def kernel(x, w_0_0, b_0_0, w_0_1, b_0_1, w_0_2, b_0_2, w_0_3, b_0_3, w_0_4, b_0_4,
           w_1_0, b_1_0, w_1_1, b_1_1, w_1_2, b_1_2, w_1_3, b_1_3, w_1_4, b_1_4,
           w_2_0, b_2_0, w_2_1, b_2_1, w_2_2, b_2_2, w_2_3, b_2_3, w_2_4, b_2_4):
    Ws = [[w_0_0, w_0_1, w_0_2, w_0_3, w_0_4],
          [w_1_0, w_1_1, w_1_2, w_1_3, w_1_4],
          [w_2_0, w_2_1, w_2_2, w_2_3, w_2_4]]
    Bs = [[b_0_0, b_0_1, b_0_2, b_0_3, b_0_4],
          [b_1_0, b_1_1, b_1_2, b_1_3, b_1_4],
          [b_2_0, b_2_1, b_2_2, b_2_3, b_2_4]]
    Wso = [Ws[2], Ws[1], Ws[0]]
    Bso = [Bs[2], Bs[1], Bs[0]]
    qB, qC, wb16 = _pools_and_weights(x, Wso, Bso)
    xa = _prep_conv0(x)
    return _run_scales(xa, qB, qC, wb16,
                       [Wso[0][3], Wso[1][3], Wso[2][3]], Bso)


# speedup vs baseline: 1.4838x; 1.0154x over previous
import functools
from types import SimpleNamespace

import numpy as np
import jax
import jax.numpy as jnp
from jax.experimental import pallas as pl
from jax.experimental.pallas import tpu as pltpu


def _lrelu(y, slope):
    return jnp.where(y >= 0.0, y, slope * y)


def _quadrant_flats(P, Hq, Wh, C):
    out = []
    for a in range(2):
        Pa = P.reshape(Hq, 2, 2 * Wh, C)[:, a]
        for b in range(2):
            Q = Pa.reshape(Hq, Wh, 2, C)[:, :, b, :]
            out.append(Q.reshape(Hq * Wh, C))
    return out


def _s2_conv_block(P, Hq, Wh, C, p_out, w_ref, b_ref, slope, out_dtype):
    qs = _quadrant_flats(P, Hq, Wh, C)
    xcat = jnp.concatenate(
        [qs[(dy % 2) * 2 + (dx % 2)][(dy // 2) * Wh + dx // 2:
                                     (dy // 2) * Wh + dx // 2 + p_out, :]
         for dy in range(4) for dx in range(4)], axis=1)
    y = jnp.dot(xcat, w_ref[...], preferred_element_type=jnp.float32)
    y = _lrelu(y + b_ref[0], slope)
    return y.astype(out_dtype)


def _scatter_pad3(dst3, y, Wi, Ho, Wo):
    dst3[...] = jnp.zeros(dst3.shape, dst3.dtype)
    for r in range(Ho):
        dst3[r + 2, 2:2 + Wo, :] = y[r * Wi:r * Wi + Wo, :]


def _scale_body(x_ref, w0, b0, w1, b1, w2, b2, w3, b3, w4, b4, o_ref,
                P1, P2, X3, X4, *, D, quad_in, slope=0.2):
    p0_out = D.Hk0 * D.Wh0
    if quad_in:
        v = x_ref[...]
        xc0 = jnp.concatenate(
            [v[g, s:s + p0_out, :] for s in (0, 1, D.Wh0, D.Wh0 + 1)
             for g in range(4)], axis=1)
        y0 = jnp.dot(xc0, w0[...], preferred_element_type=jnp.float32)
    else:
        y0 = None
        for t, s in enumerate((0, 1, D.Wh0, D.Wh0 + 1)):
            part = jnp.dot(x_ref[s:s + p0_out, :], w0[t * 12:(t + 1) * 12, :],
                           preferred_element_type=jnp.float32)
            y0 = part if y0 is None else y0 + part
    v0 = _lrelu(y0 + b0[0], slope).astype(jnp.bfloat16)

    _scatter_pad3(P1, v0, D.Wh0, D.Ho0, D.Ho0)
    v1 = _s2_conv_block(P1[...], D.Hq1, D.Wh1, 64, D.p1_out, w1, b1, slope,
                        jnp.bfloat16)

    _scatter_pad3(P2, v1, D.Wh1, D.Ho1, D.Ho1)
    v2 = _s2_conv_block(P2[...], D.Hq2, D.Wh2, 128, D.p2_out, w2, b2, slope,
                        jnp.float32)

    X3[...] = jnp.zeros(X3.shape, X3.dtype)
    for r in range(D.Ho2):
        X3[(r + 2) * D.Wi3 + 2:(r + 2) * D.Wi3 + 2 + D.Ho2, :] = \
            v2[r * D.Wh2:r * D.Wh2 + D.Ho2, :]
    x3 = X3[...]
    p3_out = D.Ho3 * D.Wi3
    shifts3 = tuple(dy * D.Wi3 + dx for dy in range(4) for dx in range(4))
    xc3 = jnp.concatenate([x3[s:s + p3_out, :] for s in shifts3], axis=1)
    y3 = jnp.dot(xc3, w3[...], preferred_element_type=jnp.float32)
    y3 = _lrelu(y3 + b3[0], slope).astype(jnp.bfloat16)

    wi4 = D.Wo3 + 4
    X4[...] = jnp.zeros(X4.shape, X4.dtype)
    for r in range(D.Ho3):
        X4[(r + 2) * wi4 + 2:(r + 2) * wi4 + 2 + D.Wo3, :] = \
            y3[r * D.Wi3:r * D.Wi3 + D.Wo3, :]
    p4_out = (D.Ho3 + 1) * wi4
    t2 = jax.lax.dot_general(w4[...], X4[...], (((1,), (1,)), ((), ())),
                             preferred_element_type=jnp.float32)
    acc4 = None
    for t, s in enumerate(dy * wi4 + dx for dy in range(4) for dx in range(4)):
        part = t2[t:t + 1, s:s + p4_out]
        acc4 = part if acc4 is None else acc4 + part
    y4 = acc4 + b4[0, 0]
    o_ref[...] = jnp.concatenate(
        [y4[:, r * wi4:r * wi4 + D.Wo4] for r in range(D.Ho4)], axis=0)


def _dims(S):
    D = SimpleNamespace()
    D.S = S
    D.Hh0 = (S + 4) // 2
    D.Wh0 = D.Hh0
    D.Hq0 = D.Hh0 + 1
    D.Hk0 = D.Hh0 - 1
    D.Ho0 = S // 2 + 1
    D.Hq1 = (D.Ho0 + 5) // 2 + 1
    D.Wh1 = (D.Ho0 + 5) // 2
    D.p1_out = (D.Wh1 - 1) * D.Wh1
    D.Ho1 = D.Ho0 // 2 + 1
    D.Hq2 = (D.Ho1 + 5) // 2 + 1
    D.Wh2 = (D.Ho1 + 5) // 2
    D.p2_out = (D.Wh2 - 1) * D.Wh2
    D.Ho2 = D.Ho1 // 2 + 1
    D.Wi3 = D.Ho2 + 4
    D.p3_in = (D.Ho2 + 5) * D.Wi3
    D.Ho3 = D.Ho2 + 1
    D.Wo3 = D.Ho2 + 1
    D.wi4 = D.Wo3 + 4
    D.p4_in = (D.Ho3 + 5) * D.wi4
    D.p4_out = (D.Ho3 + 1) * D.wi4
    D.Ho4 = D.Ho3 + 1
    D.Wo4 = D.Wo3 + 1
    return D


def _prep_conv0(x):
    N, H, W, Cin = x.shape
    xp = jnp.pad(x, ((0, 0), (2, 2), (2, 2), (0, 0)))
    Hp = xp.shape[1]
    xin = xp.reshape(N, Hp // 2, 2, Hp // 2, 2, Cin)
    xin = xin.transpose(0, 1, 3, 2, 4, 5).reshape(N, Hp // 2, Hp // 2, 4 * Cin)
    xin = jnp.pad(xin, ((0, 0), (0, 1), (0, 0), (0, 0)))
    Hh = Hp // 2
    return xin.reshape(N, (Hh + 1) * Hh, 4 * Cin).astype(jnp.bfloat16)


def _w_s2(w):
    cin, cout = w.shape[2], w.shape[3]
    return (w.reshape(2, 2, 2, 2, cin, cout)
             .transpose(0, 2, 1, 3, 4, 5)
             .reshape(16 * cin, cout))


def _pool1d(n):
    no = (n - 1) // 2 + 1
    p = np.zeros((no, n), np.float32)
    for o in range(no):
        cols = [c for c in (2 * o - 1, 2 * o, 2 * o + 1) if 0 <= c < n]
        p[o, cols] = 1.0 / len(cols)
    return p


def _quad_select_matrix(s):
    wh = (s + 4) // 2
    hq = wh + 1
    out = np.zeros((4 * hq * wh, s * s), np.float32)
    for a in range(2):
        for b in range(2):
            g = a * 2 + b
            for i in range(hq):
                for j in range(wh):
                    h, w = 2 * i + a - 2, 2 * j + b - 2
                    if 0 <= h < s and 0 <= w < s:
                        out[g * hq * wh + i * wh + j, h * s + w] = 1.0
    return out


def _pools_kernel(m1_ref, m1q_ref, m2q_ref, x_ref, *wio, N, C, pqB, pqC):
    win, wout = wio[:12], wio[14:]
    qB_ref, qC_ref = wio[12], wio[13]
    p1 = jnp.dot(m1_ref[...], x_ref[...], preferred_element_type=jnp.float32)
    p1b = p1.astype(jnp.bfloat16)
    qb = jnp.dot(m1q_ref[...], p1b,
                 preferred_element_type=jnp.float32).astype(jnp.bfloat16)
    qc = jnp.dot(m2q_ref[...], p1b,
                 preferred_element_type=jnp.float32).astype(jnp.bfloat16)
    for n in range(N):
        qB_ref[n] = qb[:, C * n:C * n + C].reshape(4, pqB, C)
        qC_ref[n] = qc[:, C * n:C * n + C].reshape(4, pqC, C)
    for i in range(12):
        wout[i][...] = win[i][...].astype(jnp.bfloat16)


def _pools_and_weights(x, Ws, Bs):
    N, H, W, C = x.shape
    m1np = np.kron(_pool1d(H), _pool1d(W))
    H2 = (H - 1) // 2 + 1
    m2np = np.kron(_pool1d(H2), _pool1d(H2))
    m1 = jnp.asarray(m1np, dtype=jnp.bfloat16)
    m1q = jnp.asarray(_quad_select_matrix(H2), dtype=jnp.bfloat16)
    m2q = jnp.asarray(_quad_select_matrix((H2 - 1) // 2 + 1) @ m2np,
                      dtype=jnp.bfloat16)
    xt = x.transpose(1, 2, 0, 3).reshape(H * W, N * C).astype(jnp.bfloat16)

    DB, DC = _dims(H2), _dims((H2 - 1) // 2 + 1)
    pqB, pqC = DB.Hq0 * DB.Wh0, DC.Hq0 * DC.Wh0

    w_f32 = []
    for k in range(3):
        ws = Ws[k]
        w_f32 += [_w_s2(ws[0]),
                  ws[1].reshape(16 * 64, 128),
                  ws[2].reshape(16 * 128, 256),
                  ws[4].reshape(16, 512)]

    out_shape = ([jax.ShapeDtypeStruct((N, 4, pqB, C), jnp.bfloat16),
                  jax.ShapeDtypeStruct((N, 4, pqC, C), jnp.bfloat16)]
                 + [jax.ShapeDtypeStruct(w.shape, jnp.bfloat16)
                    for w in w_f32])
    in_specs = ([pl.BlockSpec(m1.shape, lambda i: (0, 0)),
                 pl.BlockSpec(m1q.shape, lambda i: (0, 0)),
                 pl.BlockSpec(m2q.shape, lambda i: (0, 0)),
                 pl.BlockSpec(xt.shape, lambda i: (0, 0))]
                + [pl.BlockSpec(w.shape, lambda i: (0, 0)) for w in w_f32])
    out_specs = ([pl.BlockSpec((N, 4, pqB, C), lambda i: (0, 0, 0, 0)),
                  pl.BlockSpec((N, 4, pqC, C), lambda i: (0, 0, 0, 0))]
                 + [pl.BlockSpec(w.shape, lambda i: (0, 0)) for w in w_f32])

    outs = pl.pallas_call(
        functools.partial(_pools_kernel, N=N, C=C, pqB=pqB, pqC=pqC),
        out_shape=tuple(out_shape),
        grid=(1,),
        in_specs=in_specs,
        out_specs=tuple(out_specs),
        compiler_params=pltpu.CompilerParams(
            dimension_semantics=("arbitrary",)),
    )(m1, m1q, m2q, xt, *w_f32)
    return outs[0], outs[1], list(outs[2:])


def _tri_body(*refs, DS):
    xs = refs[0:3]
    outs = refs[33:36]
    scr = refs[36:]
    for k in range(3):
        wb = refs[3 + 10 * k:13 + 10 * k]
        _scale_body(xs[k], *wb, outs[k], *scr[4 * k:4 * k + 4],
                    D=DS[k], quad_in=(k > 0))


def _run_scales(xa_flat, qB, qC, wb16, W3s, Bs):
    N = xa_flat.shape[0]
    sizes = [64, 32, 16]
    DS = [_dims(s) for s in sizes]

    operands = [xa_flat, qB, qC]
    in_specs = [
        pl.BlockSpec((None,) + xa_flat.shape[1:], lambda n: (n, 0, 0)),
        pl.BlockSpec((None,) + qB.shape[1:], lambda n: (n, 0, 0, 0)),
        pl.BlockSpec((None,) + qC.shape[1:], lambda n: (n, 0, 0, 0)),
    ]
    for k in range(3):
        bs = Bs[k]
        packed = [wb16[4 * k + 0], bs[0].reshape(1, -1),
                  wb16[4 * k + 1], bs[1].reshape(1, -1),
                  wb16[4 * k + 2], bs[2].reshape(1, -1),
                  W3s[k].reshape(16 * 256, 512), bs[3].reshape(1, -1),
                  wb16[4 * k + 3], bs[4].reshape(1, 1)]
        for a in packed:
            operands.append(a)
            in_specs.append(pl.BlockSpec(a.shape, lambda n: (0, 0)))

    out_shapes = tuple(jax.ShapeDtypeStruct((N, D.Ho4, D.Wo4), jnp.float32)
                       for D in DS)
    out_specs = tuple(pl.BlockSpec((None, D.Ho4, D.Wo4), lambda n: (n, 0, 0))
                      for D in DS)
    scratch = []
    for D in DS:
        scratch += [pltpu.VMEM((2 * D.Hq1, 2 * D.Wh1, 64), jnp.bfloat16),
                    pltpu.VMEM((2 * D.Hq2, 2 * D.Wh2, 128), jnp.bfloat16),
                    pltpu.VMEM((D.p3_in, 256), jnp.float32),
                    pltpu.VMEM((D.p4_in, 512), jnp.bfloat16)]

    outs = pl.pallas_call(
        functools.partial(_tri_body, DS=DS),
        out_shape=out_shapes,
        grid=(N,),
        in_specs=in_specs,
        out_specs=out_specs,
        scratch_shapes=scratch,
        compiler_params=pltpu.CompilerParams(
            dimension_semantics=("parallel",)),
    )(*operands)
    return [o[..., None] for o in outs]


def kernel(x, w_0_0, b_0_0, w_0_1, b_0_1, w_0_2, b_0_2, w_0_3, b_0_3, w_0_4, b_0_4,
           w_1_0, b_1_0, w_1_1, b_1_1, w_1_2, b_1_2, w_1_3, b_1_3, w_1_4, b_1_4,
           w_2_0, b_2_0, w_2_1, b_2_1, w_2_2, b_2_2, w_2_3, b_2_3, w_2_4, b_2_4):
    Ws = [[w_0_0, w_0_1, w_0_2, w_0_3, w_0_4],
          [w_1_0, w_1_1, w_1_2, w_1_3, w_1_4],
          [w_2_0, w_2_1, w_2_2, w_2_3, w_2_4]]
    Bs = [[b_0_0, b_0_1, b_0_2, b_0_3, b_0_4],
          [b_1_0, b_1_1, b_1_2, b_1_3, b_1_4],
          [b_2_0, b_2_1, b_2_2, b_2_3, b_2_4]]
    Wso = [Ws[2], Ws[1], Ws[0]]
    Bso = [Bs[2], Bs[1], Bs[0]]
    qB, qC, wb16 = _pools_and_weights(x, Wso, Bso)
    xa = _prep_conv0(x)
    return _run_scales(xa, qB, qC, wb16,
                       [Wso[0][3], Wso[1][3], Wso[2][3]], Bso)
